# baseline (speedup 1.0000x reference)
import functools

import jax
import jax.numpy as jnp
from jax import lax
from jax.experimental import pallas as pl
from jax.experimental.pallas import tpu as pltpu
from jax.experimental.pallas import tpu_sc as plsc

B = 16384
F = 64
NC = 2
NS = 16
NW = NC * NS
BPW = B // NW
CH = 128
NCH = BPW // CH
L = 16
N_ROWS = 1000000
NPAN = (N_ROWS + 127) // 128
PAN_BASE = NPAN // NW
PAN_REM = NPAN - PAN_BASE * NW
PIECE = 2048
NPIECE = B // PIECE
XROWS = B + L
NRING = 8


def _extract_body(user_hbm, item_hbm, uft_hbm, ift_hbm, xu_hbm, xi_hbm,
                  idxbuf, cnts, starts, cursor, bucket_b, bucket_u,
                  panbuf, stage, sidx, psem, ssem):
    c = lax.axis_index("c")
    s = lax.axis_index("s")
    wid = s * NC + c
    pan0 = wid * PAN_BASE + jnp.minimum(wid, PAN_REM)
    npan = PAN_BASE + jnp.where(wid < PAN_REM, 1, 0)
    lane = lax.iota(jnp.int32, L)
    lane0 = lane == 0
    zeros = jnp.zeros((L,), jnp.int32)

    def one_table(idx_hbm, tab_hbm, out_hbm, fire_cnt0):
        for g in range(256 // L):
            cnts[pl.ds(g * L, L)] = zeros

        def scan(place):
            def piece_loop(pi, carry):
                pltpu.sync_copy(idx_hbm.at[pl.ds(pi * PIECE, PIECE)], idxbuf)

                def group_loop(g, carry2, pi=pi):
                    u16 = idxbuf[pl.ds(g * L, L)]
                    p16 = jax.lax.shift_right_logical(u16, 7)
                    m0 = (p16 >= pan0) & (p16 < pan0 + npan)

                    def cond(mv):
                        cntv = plsc.all_reduce_population_count(mv)
                        return cntv[0] > 0

                    def body(mv, u16=u16, pi=pi, g=g):
                        tv = plsc.all_reduce_ffs(mv)
                        uval = jax.lax.gather(
                            u16, tv.reshape(L, 1),
                            jax.lax.GatherDimensionNumbers(
                                offset_dims=(), collapsed_slice_dims=(0,),
                                start_index_map=(0,)),
                            (1,),
                            mode=jax.lax.GatherScatterMode.PROMISE_IN_BOUNDS)
                        pv = jax.lax.shift_right_logical(uval, 7) - pan0
                        if place:
                            bval = pi * PIECE + g * L + tv
                            pos = plsc.load_gather(cursor, [pv])
                            plsc.store_scatter(bucket_u, [pos], uval,
                                               mask=lane0)
                            plsc.store_scatter(bucket_b, [pos], bval,
                                               mask=lane0)
                            plsc.store_scatter(cursor, [pv], pos + 1,
                                               mask=lane0)
                        else:
                            c0 = plsc.load_gather(cnts, [pv])
                            plsc.store_scatter(cnts, [pv], c0 + 1, mask=lane0)
                        return mv & (lane != tv)

                    lax.while_loop(cond, body, m0)
                    return carry2

                lax.fori_loop(0, PIECE // L, group_loop, None)
                return carry

            lax.fori_loop(0, NPIECE, piece_loop, None)

        scan(place=False)

        run = zeros
        for g in range(256 // L):
            sl = pl.ds(g * L, L)
            v = cnts[sl]
            cs = plsc.cumsum(v)
            excl = cs - v + run
            starts[sl] = excl
            cursor[sl] = excl
            run = run + jnp.full((L,), cs[L - 1], jnp.int32)

        scan(place=True)

        def fire_panel(q):
            col0 = pl.multiple_of((pan0 + q) * 128, 128)
            pltpu.async_copy(
                tab_hbm.at[:, pl.ds(col0, 128)], panbuf.at[q % 2], psem)

        def wait_panel(q):
            pltpu.make_async_copy(
                tab_hbm.at[:, pl.ds(pl.multiple_of(0, 128), 128)],
                panbuf.at[q % 2], psem).wait()

        def drain_scatter():
            pltpu.make_async_copy(
                stage.at[0], out_hbm.at[sidx.at[0]], ssem).wait()

        fire_panel(0)

        def panel_loop(q, fire_cnt):
            @pl.when(q + 1 < npan)
            def _():
                fire_panel(q + 1)

            wait_panel(q)
            qv = jnp.full((L,), q, jnp.int32)
            s0 = plsc.load_gather(starts, [qv])[0]
            n0 = plsc.load_gather(cnts, [qv])[0]
            pq = panbuf.at[q % 2]

            def hit_group(h, fc, pq=pq, s0=s0, n0=n0):
                @pl.when(fc >= NRING)
                def _():
                    drain_scatter()

                slot = lax.rem(fc, NRING)
                off = s0 + h * L
                uv = bucket_u[pl.ds(off, L)]
                bv = bucket_b[pl.ds(off, L)]
                valid = (h * L + lane) < n0
                cu = uv & 127
                bsafe = jnp.where(valid, bv, B)
                st = stage.at[slot]
                for cc in range(F):
                    vals = plsc.load_gather(
                        pq, [jnp.full((L,), cc, jnp.int32), cu])
                    plsc.store_scatter(st, [lane, jnp.full((L,), cc, jnp.int32)],
                                       vals)
                sidx[slot, :] = bsafe
                pltpu.async_copy(st, out_hbm.at[sidx.at[slot]], ssem)
                return fc + 1

            nh = lax.div(n0 + L - 1, L)
            return lax.fori_loop(0, nh, hit_group, fire_cnt)

        fire_cnt = lax.fori_loop(0, npan, panel_loop, fire_cnt0)
        return fire_cnt

    fc = one_table(user_hbm, uft_hbm, xu_hbm, jnp.int32(0))
    fc = one_table(item_hbm, ift_hbm, xi_hbm, fc)

    def drain_rest(i, carry):
        pltpu.make_async_copy(
            stage.at[0], xi_hbm.at[sidx.at[0]], ssem).wait()
        return carry

    lax.fori_loop(0, jnp.minimum(fc, NRING), drain_rest, None)


def _dot_body(user_hbm, item_hbm, xu_hbm, xi_hbm, ub_hbm, ib_hbm, out_hbm,
              uidx, iidx, urows, irows, ubias, ibias, outv, sem):
    c = lax.axis_index("c")
    s = lax.axis_index("s")
    wid = s * NC + c
    row0 = wid * BPW

    pltpu.sync_copy(user_hbm.at[pl.ds(wid * 2, 2)], uidx)
    pltpu.sync_copy(item_hbm.at[pl.ds(wid * 2, 2)], iidx)

    r0 = pl.multiple_of(row0, 8)
    copies = []
    for k in range(NCH):
        dst = pl.ds(k * CH, CH)
        copies.append(pltpu.async_copy(ub_hbm.at[uidx.at[k // 2, k % 2]], ubias.at[dst], sem))
        copies.append(pltpu.async_copy(ib_hbm.at[iidx.at[k // 2, k % 2]], ibias.at[dst], sem))

    lane = lax.iota(jnp.int32, L)

    for k in range(NCH):
        kb = k % 2
        pltpu.sync_copy(xu_hbm.at[pl.ds(r0 + k * CH, CH)], urows.at[kb])
        pltpu.sync_copy(xi_hbm.at[pl.ds(r0 + k * CH, CH)], irows.at[kb])
        uk = urows.at[kb]
        ik = irows.at[kb]

        def grp(g, carry, uk=uk, ik=ik, k=k):
            base = g * L
            res = jnp.zeros((L,), jnp.float32)
            for t in range(L):
                r = base + t
                acc = uk[r, pl.ds(0, L)] * ik[r, pl.ds(0, L)]
                for q in range(1, F // L):
                    acc = acc + uk[r, pl.ds(q * L, L)] * ik[r, pl.ds(q * L, L)]
                res = jnp.where(lane == t, jnp.sum(acc), res)
            outv[pl.ds(k * CH + base, L)] = res
            return carry

        lax.fori_loop(0, CH // L, grp, None)

    for cp in copies:
        cp.wait()
    for m in range(BPW // L):
        sl = pl.ds(m * L, L)
        outv[sl] = outv[sl] + ubias[sl] + ibias[sl]
    pltpu.sync_copy(outv, out_hbm.at[pl.ds(r0, BPW)])


@jax.jit
def _call(user1, item1, user2, item2, uft, ift, ub, ib):
    mesh = plsc.VectorSubcoreMesh(core_axis_name="c", subcore_axis_name="s")
    params = pltpu.CompilerParams(
        needs_layout_passes=False, use_tc_tiling_on_sc=True)

    extract = functools.partial(
        pl.kernel,
        out_type=(jax.ShapeDtypeStruct((XROWS, 128), jnp.float32),
                  jax.ShapeDtypeStruct((XROWS, 128), jnp.float32)),
        mesh=mesh,
        compiler_params=params,
        scratch_types=[
            pltpu.VMEM((PIECE,), jnp.int32),
            pltpu.VMEM((256,), jnp.int32),
            pltpu.VMEM((256,), jnp.int32),
            pltpu.VMEM((256,), jnp.int32),
            pltpu.VMEM((B + L,), jnp.int32),
            pltpu.VMEM((B + L,), jnp.int32),
            pltpu.VMEM((2, F, 128), jnp.float32),
            pltpu.VMEM((NRING, L, 128), jnp.float32),
            pltpu.VMEM((NRING, L), jnp.int32),
            pltpu.SemaphoreType.DMA,
            pltpu.SemaphoreType.DMA,
        ],
    )
    xu, xi = extract(_extract_body)(user1, item1, uft, ift)

    dot = functools.partial(
        pl.kernel,
        out_type=jax.ShapeDtypeStruct((B,), jnp.float32),
        mesh=mesh,
        compiler_params=params,
        scratch_types=[
            pltpu.VMEM((2, 2, CH), jnp.int32),
            pltpu.VMEM((2, 2, CH), jnp.int32),
            pltpu.VMEM((2, CH, 128), jnp.float32),
            pltpu.VMEM((2, CH, 128), jnp.float32),
            pltpu.VMEM((BPW,), jnp.float32),
            pltpu.VMEM((BPW,), jnp.float32),
            pltpu.VMEM((BPW,), jnp.float32),
            pltpu.SemaphoreType.DMA,
        ],
    )
    return dot(_dot_body)(user2, item2, xu, xi, ub, ib)


def kernel(user, item, user_factors, item_factors, users_biases, items_biases):
    user1 = user.astype(jnp.int32)
    item1 = item.astype(jnp.int32)
    user2 = user1.reshape(B // CH // 2, 2, CH)
    item2 = item1.reshape(B // CH // 2, 2, CH)
    return _call(user1, item1, user2, item2,
                 user_factors.T, item_factors.T,
                 users_biases.reshape(-1), items_biases.reshape(-1))


# speedup vs baseline: 7.0616x; 7.0616x over previous
import functools

import jax
import jax.numpy as jnp
from jax import lax
from jax.experimental import pallas as pl
from jax.experimental.pallas import tpu as pltpu
from jax.experimental.pallas import tpu_sc as plsc

B = 16384
F = 64
NC = 2
NS = 16
NW = NC * NS
BPW = B // NW
CH = 128
NCH = BPW // CH
L = 16


def _body(user_hbm, item_hbm, uf_hbm, if_hbm, ub_hbm, ib_hbm, out_hbm,
          uidx, iidx, urows, irows, ubias, ibias, outv, sem):
    c = lax.axis_index("c")
    s = lax.axis_index("s")
    wid = s * NC + c
    row0 = wid * NCH

    pltpu.sync_copy(user_hbm.at[pl.ds(row0, NCH)], uidx)
    pltpu.sync_copy(item_hbm.at[pl.ds(row0, NCH)], iidx)

    copies = []
    for k in range(NCH):
        dst = pl.ds(k * CH, CH)
        copies.append(pltpu.async_copy(uf_hbm.at[uidx.at[k]], urows.at[dst], sem))
        copies.append(pltpu.async_copy(if_hbm.at[iidx.at[k]], irows.at[dst], sem))
        copies.append(pltpu.async_copy(ub_hbm.at[uidx.at[k]], ubias.at[dst], sem))
        copies.append(pltpu.async_copy(ib_hbm.at[iidx.at[k]], ibias.at[dst], sem))
    for cp in copies:
        cp.wait()

    lane = lax.iota(jnp.int32, L)

    def grp(g, carry):
        base = g * L
        res = jnp.zeros((L,), jnp.float32)
        for t in range(L):
            r = base + t
            acc = urows[r, pl.ds(0, L)] * irows[r, pl.ds(0, L)]
            for q in range(1, F // L):
                acc = acc + urows[r, pl.ds(q * L, L)] * irows[r, pl.ds(q * L, L)]
            res = jnp.where(lane == t, jnp.sum(acc), res)
        sl = pl.ds(base, L)
        outv[sl] = res + ubias[sl] + ibias[sl]
        return carry

    lax.fori_loop(0, BPW // L, grp, None)

    pltpu.sync_copy(outv, out_hbm.at[pl.ds(wid * BPW, BPW)])


@jax.jit
def _sc_call(user2, item2, uf, itf, ub, ib):
    grid_kernel = functools.partial(
        pl.kernel,
        out_type=jax.ShapeDtypeStruct((B,), jnp.float32),
        mesh=plsc.VectorSubcoreMesh(core_axis_name="c", subcore_axis_name="s"),
        compiler_params=pltpu.CompilerParams(
            needs_layout_passes=False, use_tc_tiling_on_sc=False),
        scratch_types=[
            pltpu.VMEM((NCH, CH), jnp.int32),
            pltpu.VMEM((NCH, CH), jnp.int32),
            pltpu.VMEM((BPW, F), jnp.float32),
            pltpu.VMEM((BPW, F), jnp.float32),
            pltpu.VMEM((BPW,), jnp.float32),
            pltpu.VMEM((BPW,), jnp.float32),
            pltpu.VMEM((BPW,), jnp.float32),
            pltpu.SemaphoreType.DMA,
        ],
    )
    return grid_kernel(_body)(user2, item2, uf, itf, ub, ib)


def kernel(user, item, user_factors, item_factors, users_biases, items_biases):
    user2 = user.astype(jnp.int32).reshape(B // CH, CH)
    item2 = item.astype(jnp.int32).reshape(B // CH, CH)
    ub = users_biases.reshape(-1)
    ib = items_biases.reshape(-1)
    return _sc_call(user2, item2, user_factors, item_factors, ub, ib)
